# baseline (speedup 1.0000x reference)
import functools

import jax
import jax.numpy as jnp
from jax import lax
from jax.experimental import pallas as pl
from jax.experimental.pallas import tpu as pltpu
from jax.experimental.pallas import tpu_sc as plsc

N = 10000
M = 32
F = 128
BD = 16
E = N * M
F2 = 2 * F

NB = 200
EB = NB * M
NBLK = N // NB

_EPS = 1e-5


_SC_CHUNK = 80


def _sc_gather(idx_flat, table):
    info = plsc.get_sparse_core_info()
    nw = info.num_cores * info.num_subcores
    per_w = E // nw
    n_ch = per_w // _SC_CHUNK
    n_pair = n_ch // 2
    mesh = plsc.VectorSubcoreMesh(core_axis_name="c", subcore_axis_name="s")

    @functools.partial(
        pl.kernel,
        out_type=jax.ShapeDtypeStruct((E, F), jnp.float32),
        mesh=mesh,
        scratch_types=[
            pltpu.VMEM((per_w,), jnp.int32),
            pltpu.VMEM((_SC_CHUNK, F), jnp.float32),
            pltpu.VMEM((_SC_CHUNK, F), jnp.float32),
            pltpu.SemaphoreType.DMA,
            pltpu.SemaphoreType.DMA,
            pltpu.SemaphoreType.DMA,
            pltpu.SemaphoreType.DMA,
        ],
    )
    def gk(idx_hbm, table_hbm, out_hbm, idx_v, rows0, rows1,
           sg0, sg1, ss0, ss1):
        wid = lax.axis_index("s") * info.num_cores + lax.axis_index("c")
        base = wid * per_w
        rows = (rows0, rows1)
        sg = (sg0, sg1)
        ss = (ss0, ss1)

        def gather(c, p):
            pltpu.async_copy(
                table_hbm.at[idx_v.at[pl.ds(c * _SC_CHUNK, _SC_CHUNK)]],
                rows[p], sg[p])

        def store(c, p):
            pltpu.async_copy(
                rows[p], out_hbm.at[pl.ds(base + c * _SC_CHUNK, _SC_CHUNK)],
                ss[p])

        def wait_gather(p):
            pltpu.make_async_copy(
                table_hbm.at[pl.ds(0, _SC_CHUNK)], rows[p], sg[p]).wait()

        def wait_store(p):
            pltpu.make_async_copy(
                rows[p], out_hbm.at[pl.ds(base, _SC_CHUNK)], ss[p]).wait()

        pltpu.sync_copy(idx_hbm.at[pl.ds(base, per_w)], idx_v)
        gather(0, 0)

        @pl.loop(0, n_pair)
        def _(i):
            c0 = 2 * i
            wait_gather(0)
            store(c0, 0)

            @pl.when(i > 0)
            def _():
                wait_store(1)

            gather(c0 + 1, 1)
            wait_gather(1)
            store(c0 + 1, 1)

            @pl.when(c0 + 2 < n_ch)
            def _():
                wait_store(0)
                gather(c0 + 2, 0)

        if n_ch % 2:
            wait_gather(0)
            store(n_ch - 1, 0)
            wait_store(1)
            wait_store(0)
        else:
            wait_store(1)

    return gk(idx_flat, table)


def _edge_z(atom_ref, g_ref, bond_ref, ws, wn, wb, shift=None):
    hi = jax.lax.Precision.HIGHEST
    s1 = jnp.dot(atom_ref[...], ws, preferred_element_type=jnp.float32,
                 precision=hi)
    if shift is not None:
        s1 = s1 + shift
    zg = jnp.dot(g_ref[...], wn, preferred_element_type=jnp.float32,
                 precision=hi)
    zq = jnp.dot(bond_ref[...], wb, preferred_element_type=jnp.float32,
                 precision=hi)
    return (zg + zq).reshape(NB, M, F2) + s1[:, None, :]


def _stats_body(atom_ref, g_ref, bond_ref, ws_ref, wn_ref, wb_ref, out_ref):
    z = _edge_z(atom_ref, g_ref, bond_ref, ws_ref[...], wn_ref[...],
                wb_ref[...])
    ps = jnp.sum(z, axis=(0, 1))
    psq = jnp.sum(z * z, axis=(0, 1))
    acc = jnp.concatenate([ps[None, :], psq[None, :]], axis=0)

    @pl.when(pl.program_id(0) == 0)
    def _():
        out_ref[...] = jnp.zeros_like(out_ref)

    out_ref[...] += acc


def _softplus(x):
    return jnp.maximum(x, 0.0) + jnp.log(1.0 + jnp.exp(-jnp.abs(x)))


def _apply_body(st_ref, atom_ref, g_ref, bond_ref, ws_ref, wn_ref, wb_ref,
                sc1_ref, of1_ref, s_ref, st2_ref):
    st = st_ref[...]
    mean = st[0:1, :] * (1.0 / E)
    var = st[1:2, :] * (1.0 / E) - mean * mean
    inv = sc1_ref[...] * lax.rsqrt(var + _EPS)
    beta = of1_ref[...] - mean * inv

    z = _edge_z(atom_ref, g_ref, bond_ref, ws_ref[...], wn_ref[...],
                wb_ref[...])
    y = z * inv[None, :, :] + beta[None, :, :]
    filt = 1.0 / (1.0 + jnp.exp(-y[:, :, :F]))
    core = _softplus(y[:, :, F:])
    s_nb = jnp.sum(filt * core, axis=1)
    s_ref[...] = s_nb
    acc = jnp.concatenate([jnp.sum(s_nb, axis=0)[None, :],
                           jnp.sum(s_nb * s_nb, axis=0)[None, :]], axis=0)

    @pl.when(pl.program_id(0) == 0)
    def _():
        st2_ref[...] = jnp.zeros_like(st2_ref)

    st2_ref[...] += acc


def _final_body(st2_ref, atom_ref, s_ref, sc2_ref, of2_ref, out_ref):
    st = st2_ref[...]
    mean = st[0:1, :] * (1.0 / N)
    var = st[1:2, :] * (1.0 / N) - mean * mean
    inv = sc2_ref[...] * lax.rsqrt(var + _EPS)
    beta = of2_ref[...] - mean * inv
    out_ref[...] = _softplus(atom_ref[...] + s_ref[...] * inv + beta)


def _full(shape):
    return pl.BlockSpec(shape, lambda i: (0,) * len(shape))


def _stats_call(atom, g, bond2d, ws, wn, wb):
    return pl.pallas_call(
        _stats_body,
        grid=(NBLK,),
        in_specs=[
            pl.BlockSpec((NB, F), lambda i: (i, 0)),
            pl.BlockSpec((EB, F), lambda i: (i, 0)),
            pl.BlockSpec((EB, BD), lambda i: (i, 0)),
            _full((F, F2)),
            _full((F, F2)),
            _full((BD, F2)),
        ],
        out_specs=_full((2, F2)),
        out_shape=jax.ShapeDtypeStruct((2, F2), jnp.float32),
        compiler_params=pltpu.CompilerParams(
            dimension_semantics=("arbitrary",)),
    )(atom, g, bond2d, ws, wn, wb)


def _apply_call(st1, atom, g, bond2d, ws, wn, wb, sc1, of1):
    return pl.pallas_call(
        _apply_body,
        grid=(NBLK,),
        in_specs=[
            _full((2, F2)),
            pl.BlockSpec((NB, F), lambda i: (i, 0)),
            pl.BlockSpec((EB, F), lambda i: (i, 0)),
            pl.BlockSpec((EB, BD), lambda i: (i, 0)),
            _full((F, F2)),
            _full((F, F2)),
            _full((BD, F2)),
            _full((1, F2)),
            _full((1, F2)),
        ],
        out_specs=[
            pl.BlockSpec((NB, F), lambda i: (i, 0)),
            _full((2, F)),
        ],
        out_shape=[
            jax.ShapeDtypeStruct((N, F), jnp.float32),
            jax.ShapeDtypeStruct((2, F), jnp.float32),
        ],
        compiler_params=pltpu.CompilerParams(
            dimension_semantics=("arbitrary",)),
    )(st1, atom, g, bond2d, ws, wn, wb, sc1, of1)


def _final_call(st2, atom, s, sc2, of2):
    return pl.pallas_call(
        _final_body,
        grid=(NBLK,),
        in_specs=[
            _full((2, F)),
            pl.BlockSpec((NB, F), lambda i: (i, 0)),
            pl.BlockSpec((NB, F), lambda i: (i, 0)),
            _full((1, F)),
            _full((1, F)),
        ],
        out_specs=pl.BlockSpec((NB, F), lambda i: (i, 0)),
        out_shape=jax.ShapeDtypeStruct((N, F), jnp.float32),
        compiler_params=pltpu.CompilerParams(
            dimension_semantics=("arbitrary",)),
    )(st2, atom, s, sc2, of2)


def kernel(neighbor_indices, atom_features, bond_features, W, b,
           bn1_scale, bn1_offset, bn2_scale, bn2_offset):
    del b
    idx_flat = neighbor_indices.reshape(E).astype(jnp.int32)
    bond2d = bond_features.reshape(E, BD)
    ws, wn, wb = W[:F], W[F:2 * F], W[2 * F:]

    g = _sc_gather(idx_flat, atom_features)
    st1 = _stats_call(atom_features, g, bond2d, ws, wn, wb)
    s, st2 = _apply_call(st1, atom_features, g, bond2d, ws, wn, wb,
                         bn1_scale.reshape(1, F2), bn1_offset.reshape(1, F2))
    return _final_call(st2, atom_features, s,
                       bn2_scale.reshape(1, F), bn2_offset.reshape(1, F))


# speedup vs baseline: 2.6836x; 2.6836x over previous
import functools

import jax
import jax.numpy as jnp
from jax import lax
from jax.experimental import pallas as pl
from jax.experimental.pallas import tpu as pltpu
from jax.experimental.pallas import tpu_sc as plsc

N = 10000
M = 32
F = 128
BD = 16
E = N * M
F2 = 2 * F

NB = 200
EB = NB * M
NBLK = N // NB

_EPS = 1e-5


_SC_CHUNK = 80


def _sc_gather(idx_flat, table):
    info = plsc.get_sparse_core_info()
    nw = info.num_cores * info.num_subcores
    per_w = E // nw
    n_ch = per_w // _SC_CHUNK
    n_pair = n_ch // 2
    mesh = plsc.VectorSubcoreMesh(core_axis_name="c", subcore_axis_name="s")

    @functools.partial(
        pl.kernel,
        out_type=jax.ShapeDtypeStruct((E, F), jnp.float32),
        mesh=mesh,
        scratch_types=[
            pltpu.VMEM((per_w,), jnp.int32),
            pltpu.VMEM((_SC_CHUNK, F), jnp.float32),
            pltpu.VMEM((_SC_CHUNK, F), jnp.float32),
            pltpu.SemaphoreType.DMA,
            pltpu.SemaphoreType.DMA,
            pltpu.SemaphoreType.DMA,
            pltpu.SemaphoreType.DMA,
        ],
    )
    def gk(idx_hbm, table_hbm, out_hbm, idx_v, rows0, rows1,
           sg0, sg1, ss0, ss1):
        wid = lax.axis_index("s") * info.num_cores + lax.axis_index("c")
        base = wid * per_w
        rows = (rows0, rows1)
        sg = (sg0, sg1)
        ss = (ss0, ss1)

        def gather(c, p):
            pltpu.async_copy(
                table_hbm.at[idx_v.at[pl.ds(c * _SC_CHUNK, _SC_CHUNK)]],
                rows[p], sg[p])

        def store(c, p):
            pltpu.async_copy(
                rows[p], out_hbm.at[pl.ds(base + c * _SC_CHUNK, _SC_CHUNK)],
                ss[p])

        def wait_gather(p):
            pltpu.make_async_copy(
                table_hbm.at[pl.ds(0, _SC_CHUNK)], rows[p], sg[p]).wait()

        def wait_store(p):
            pltpu.make_async_copy(
                rows[p], out_hbm.at[pl.ds(base, _SC_CHUNK)], ss[p]).wait()

        pltpu.sync_copy(idx_hbm.at[pl.ds(base, per_w)], idx_v)
        gather(0, 0)

        @pl.loop(0, n_pair)
        def _(i):
            c0 = 2 * i
            wait_gather(0)
            store(c0, 0)

            @pl.when(i > 0)
            def _():
                wait_store(1)

            gather(c0 + 1, 1)
            wait_gather(1)
            store(c0 + 1, 1)

            @pl.when(c0 + 2 < n_ch)
            def _():
                wait_store(0)
                gather(c0 + 2, 0)

        if n_ch % 2:
            wait_gather(0)
            store(n_ch - 1, 0)
            wait_store(1)
            wait_store(0)
        else:
            wait_store(1)

    return gk(idx_flat, table)


def _edge_z(atom_ref, g_ref, bond_ref, ws, wn, wb, shift=None):
    s1 = jnp.dot(atom_ref[...], ws, preferred_element_type=jnp.float32)
    if shift is not None:
        s1 = s1 + shift
    zg = jnp.dot(g_ref[...], wn, preferred_element_type=jnp.float32)
    zq = jnp.dot(bond_ref[...], wb, preferred_element_type=jnp.float32)
    return (zg + zq).reshape(NB, M, F2) + s1[:, None, :]


def _stats_body(atom_ref, g_ref, bond_ref, ws_ref, wn_ref, wb_ref, out_ref):
    z = _edge_z(atom_ref, g_ref, bond_ref, ws_ref[...], wn_ref[...],
                wb_ref[...])
    ps = jnp.sum(z, axis=(0, 1))
    psq = jnp.sum(z * z, axis=(0, 1))
    acc = jnp.concatenate([ps[None, :], psq[None, :]], axis=0)

    @pl.when(pl.program_id(0) == 0)
    def _():
        out_ref[...] = jnp.zeros_like(out_ref)

    out_ref[...] += acc


def _softplus(x):
    return jnp.maximum(x, 0.0) + jnp.log(1.0 + jnp.exp(-jnp.abs(x)))


def _apply_body(st_ref, atom_ref, g_ref, bond_ref, ws_ref, wn_ref, wb_ref,
                sc1_ref, of1_ref, s_ref, st2_ref):
    st = st_ref[...]
    mean = st[0:1, :] * (1.0 / E)
    var = st[1:2, :] * (1.0 / E) - mean * mean
    inv = sc1_ref[...] * lax.rsqrt(var + _EPS)
    beta = of1_ref[...] - mean * inv

    z = _edge_z(atom_ref, g_ref, bond_ref, ws_ref[...], wn_ref[...],
                wb_ref[...])
    y = z * inv[None, :, :] + beta[None, :, :]
    filt = 1.0 / (1.0 + jnp.exp(-y[:, :, :F]))
    core = _softplus(y[:, :, F:])
    s_nb = jnp.sum(filt * core, axis=1)
    s_ref[...] = s_nb
    acc = jnp.concatenate([jnp.sum(s_nb, axis=0)[None, :],
                           jnp.sum(s_nb * s_nb, axis=0)[None, :]], axis=0)

    @pl.when(pl.program_id(0) == 0)
    def _():
        st2_ref[...] = jnp.zeros_like(st2_ref)

    st2_ref[...] += acc


def _final_body(st2_ref, atom_ref, s_ref, sc2_ref, of2_ref, out_ref):
    st = st2_ref[...]
    mean = st[0:1, :] * (1.0 / N)
    var = st[1:2, :] * (1.0 / N) - mean * mean
    inv = sc2_ref[...] * lax.rsqrt(var + _EPS)
    beta = of2_ref[...] - mean * inv
    out_ref[...] = _softplus(atom_ref[...] + s_ref[...] * inv + beta)


def _full(shape):
    return pl.BlockSpec(shape, lambda i: (0,) * len(shape))


def _stats_call(atom, g, bond2d, ws, wn, wb):
    return pl.pallas_call(
        _stats_body,
        grid=(NBLK,),
        in_specs=[
            pl.BlockSpec((NB, F), lambda i: (i, 0)),
            pl.BlockSpec((EB, F), lambda i: (i, 0)),
            pl.BlockSpec((EB, BD), lambda i: (i, 0)),
            _full((F, F2)),
            _full((F, F2)),
            _full((BD, F2)),
        ],
        out_specs=_full((2, F2)),
        out_shape=jax.ShapeDtypeStruct((2, F2), jnp.float32),
        compiler_params=pltpu.CompilerParams(
            dimension_semantics=("arbitrary",)),
    )(atom, g, bond2d, ws, wn, wb)


def _apply_call(st1, atom, g, bond2d, ws, wn, wb, sc1, of1):
    return pl.pallas_call(
        _apply_body,
        grid=(NBLK,),
        in_specs=[
            _full((2, F2)),
            pl.BlockSpec((NB, F), lambda i: (i, 0)),
            pl.BlockSpec((EB, F), lambda i: (i, 0)),
            pl.BlockSpec((EB, BD), lambda i: (i, 0)),
            _full((F, F2)),
            _full((F, F2)),
            _full((BD, F2)),
            _full((1, F2)),
            _full((1, F2)),
        ],
        out_specs=[
            pl.BlockSpec((NB, F), lambda i: (i, 0)),
            _full((2, F)),
        ],
        out_shape=[
            jax.ShapeDtypeStruct((N, F), jnp.float32),
            jax.ShapeDtypeStruct((2, F), jnp.float32),
        ],
        compiler_params=pltpu.CompilerParams(
            dimension_semantics=("arbitrary",)),
    )(st1, atom, g, bond2d, ws, wn, wb, sc1, of1)


def _final_call(st2, atom, s, sc2, of2):
    return pl.pallas_call(
        _final_body,
        grid=(NBLK,),
        in_specs=[
            _full((2, F)),
            pl.BlockSpec((NB, F), lambda i: (i, 0)),
            pl.BlockSpec((NB, F), lambda i: (i, 0)),
            _full((1, F)),
            _full((1, F)),
        ],
        out_specs=pl.BlockSpec((NB, F), lambda i: (i, 0)),
        out_shape=jax.ShapeDtypeStruct((N, F), jnp.float32),
        compiler_params=pltpu.CompilerParams(
            dimension_semantics=("arbitrary",)),
    )(st2, atom, s, sc2, of2)


def kernel(neighbor_indices, atom_features, bond_features, W, b,
           bn1_scale, bn1_offset, bn2_scale, bn2_offset):
    del b
    idx_flat = neighbor_indices.reshape(E).astype(jnp.int32)
    bond2d = bond_features.reshape(E, BD)
    ws, wn, wb = W[:F], W[F:2 * F], W[2 * F:]

    g = _sc_gather(idx_flat, atom_features)
    st1 = _stats_call(atom_features, g, bond2d, ws, wn, wb)
    s, st2 = _apply_call(st1, atom_features, g, bond2d, ws, wn, wb,
                         bn1_scale.reshape(1, F2), bn1_offset.reshape(1, F2))
    return _final_call(st2, atom_features, s,
                       bn2_scale.reshape(1, F), bn2_offset.reshape(1, F))


# speedup vs baseline: 2.7533x; 1.0260x over previous
import functools

import jax
import jax.numpy as jnp
from jax import lax
from jax.experimental import pallas as pl
from jax.experimental.pallas import tpu as pltpu
from jax.experimental.pallas import tpu_sc as plsc

N = 10000
M = 32
F = 128
BD = 16
E = N * M
F2 = 2 * F

NB = 200
EB = NB * M
NBLK = N // NB

_EPS = 1e-5


_SC_CHUNK = 80


def _sc_gather(idx_flat, table):
    info = plsc.get_sparse_core_info()
    nw = info.num_cores * info.num_subcores
    per_w = E // nw
    n_ch = per_w // _SC_CHUNK
    n_pair = n_ch // 2
    mesh = plsc.VectorSubcoreMesh(core_axis_name="c", subcore_axis_name="s")

    @functools.partial(
        pl.kernel,
        out_type=jax.ShapeDtypeStruct((E, F), jnp.float32),
        mesh=mesh,
        scratch_types=[
            pltpu.VMEM((per_w,), jnp.int32),
            pltpu.VMEM((_SC_CHUNK, F), jnp.float32),
            pltpu.VMEM((_SC_CHUNK, F), jnp.float32),
            pltpu.SemaphoreType.DMA,
            pltpu.SemaphoreType.DMA,
            pltpu.SemaphoreType.DMA,
            pltpu.SemaphoreType.DMA,
        ],
    )
    def gk(idx_hbm, table_hbm, out_hbm, idx_v, rows0, rows1,
           sg0, sg1, ss0, ss1):
        wid = lax.axis_index("s") * info.num_cores + lax.axis_index("c")
        base = wid * per_w
        rows = (rows0, rows1)
        sg = (sg0, sg1)
        ss = (ss0, ss1)

        def gather(c, p):
            pltpu.async_copy(
                table_hbm.at[idx_v.at[pl.ds(c * _SC_CHUNK, _SC_CHUNK)]],
                rows[p], sg[p])

        def store(c, p):
            pltpu.async_copy(
                rows[p], out_hbm.at[pl.ds(base + c * _SC_CHUNK, _SC_CHUNK)],
                ss[p])

        def wait_gather(p):
            pltpu.make_async_copy(
                table_hbm.at[pl.ds(0, _SC_CHUNK)], rows[p], sg[p]).wait()

        def wait_store(p):
            pltpu.make_async_copy(
                rows[p], out_hbm.at[pl.ds(base, _SC_CHUNK)], ss[p]).wait()

        pltpu.sync_copy(idx_hbm.at[pl.ds(base, per_w)], idx_v)
        gather(0, 0)

        @pl.loop(0, n_pair)
        def _(i):
            c0 = 2 * i
            wait_gather(0)
            store(c0, 0)

            @pl.when(i > 0)
            def _():
                wait_store(1)

            gather(c0 + 1, 1)
            wait_gather(1)
            store(c0 + 1, 1)

            @pl.when(c0 + 2 < n_ch)
            def _():
                wait_store(0)
                gather(c0 + 2, 0)

        if n_ch % 2:
            wait_gather(0)
            store(n_ch - 1, 0)
            wait_store(1)
            wait_store(0)
        else:
            wait_store(1)

    return gk(idx_flat, table)


def _edge_z(atom_ref, g_ref, bond_ref, ws_ref, wn_ref, wb_ref):
    s1 = jnp.dot(atom_ref[...], ws_ref[...],
                 preferred_element_type=jnp.float32)
    zg = jnp.dot(g_ref[...], wn_ref[...], preferred_element_type=jnp.float32)
    zq = jnp.dot(bond_ref[...].reshape(EB, BD), wb_ref[...],
                 preferred_element_type=jnp.float32)
    return (zg + zq).reshape(NB, M, F2) + s1[:, None, :]


def _softplus(x):
    return jnp.maximum(x, 0.0) + jnp.log(1.0 + jnp.exp(-jnp.abs(x)))


def _fused_body(atom_ref, g_ref, bond_ref, ws_ref, wn_ref, wb_ref,
                sc1_ref, of1_ref, sc2_ref, of2_ref,
                out_ref, st1_ref, st2_ref, s_ref):
    p = pl.program_id(0)
    i = pl.program_id(1)

    @pl.when(p == 0)
    def _():
        z = _edge_z(atom_ref, g_ref, bond_ref, ws_ref, wn_ref, wb_ref)
        acc = jnp.concatenate([jnp.sum(z, axis=(0, 1))[None, :],
                               jnp.sum(z * z, axis=(0, 1))[None, :]], axis=0)

        @pl.when(i == 0)
        def _():
            st1_ref[...] = jnp.zeros_like(st1_ref)

        st1_ref[...] += acc

    @pl.when(p == 1)
    def _():
        st = st1_ref[...]
        mean = st[0:1, :] * (1.0 / E)
        var = st[1:2, :] * (1.0 / E) - mean * mean
        inv = sc1_ref[...] * lax.rsqrt(var + _EPS)
        beta = of1_ref[...] - mean * inv

        z = _edge_z(atom_ref, g_ref, bond_ref, ws_ref, wn_ref, wb_ref)
        y = z * inv[None, :, :] + beta[None, :, :]
        filt = 1.0 / (1.0 + jnp.exp(-y[:, :, :F]))
        core = _softplus(y[:, :, F:])
        s_nb = jnp.sum(filt * core, axis=1)
        s_ref[pl.ds(i * NB, NB), :] = s_nb
        acc = jnp.concatenate([jnp.sum(s_nb, axis=0)[None, :],
                               jnp.sum(s_nb * s_nb, axis=0)[None, :]], axis=0)

        @pl.when(i == 0)
        def _():
            st2_ref[...] = jnp.zeros_like(st2_ref)

        st2_ref[...] += acc

    @pl.when(p == 2)
    def _():
        st = st2_ref[...]
        mean = st[0:1, :] * (1.0 / N)
        var = st[1:2, :] * (1.0 / N) - mean * mean
        inv = sc2_ref[...] * lax.rsqrt(var + _EPS)
        beta = of2_ref[...] - mean * inv
        out_ref[...] = _softplus(
            atom_ref[...] + s_ref[pl.ds(i * NB, NB), :] * inv + beta)


def _full(shape):
    return pl.BlockSpec(shape, lambda p, i: (0,) * len(shape))


def _fused_call(atom, g, bond, ws, wn, wb, sc1, of1, sc2, of2):
    def edge_map(p, i):
        return (jnp.where(p == 2, 0, i), 0)

    def edge_map3(p, i):
        return (jnp.where(p == 2, 0, i), 0, 0)

    return pl.pallas_call(
        _fused_body,
        grid=(3, NBLK),
        in_specs=[
            pl.BlockSpec((NB, F), lambda p, i: (i, 0)),
            pl.BlockSpec((EB, F), edge_map),
            pl.BlockSpec((NB, M, BD), edge_map3),
            _full((F, F2)),
            _full((F, F2)),
            _full((BD, F2)),
            _full((1, F2)),
            _full((1, F2)),
            _full((1, F)),
            _full((1, F)),
        ],
        out_specs=pl.BlockSpec((NB, F), lambda p, i: (i, 0)),
        out_shape=jax.ShapeDtypeStruct((N, F), jnp.float32),
        scratch_shapes=[
            pltpu.VMEM((2, F2), jnp.float32),
            pltpu.VMEM((2, F), jnp.float32),
            pltpu.VMEM((N, F), jnp.float32),
        ],
        compiler_params=pltpu.CompilerParams(
            dimension_semantics=("arbitrary", "arbitrary")),
    )(atom, g, bond, ws, wn, wb, sc1, of1, sc2, of2)


def kernel(neighbor_indices, atom_features, bond_features, W, b,
           bn1_scale, bn1_offset, bn2_scale, bn2_offset):
    del b
    idx_flat = neighbor_indices.reshape(E).astype(jnp.int32)
    ws, wn, wb = W[:F], W[F:2 * F], W[2 * F:]

    g = _sc_gather(idx_flat, atom_features)
    return _fused_call(atom_features, g, bond_features, ws, wn, wb,
                       bn1_scale.reshape(1, F2), bn1_offset.reshape(1, F2),
                       bn2_scale.reshape(1, F), bn2_offset.reshape(1, F))
